# Initial kernel scaffold; baseline (speedup 1.0000x reference)
import jax
import jax.numpy as jnp
from jax.experimental import pallas as pl


def kernel(inputs):
    raise NotImplementedError("write your pallas kernel here")


# speedup vs baseline: 1.4005x; 1.4005x over previous
import functools

import jax
import jax.numpy as jnp
import numpy as np
from jax import lax
from jax.experimental import pallas as pl
from jax.experimental.pallas import tpu as pltpu
from jax.experimental.pallas import tpu_sc as plsc

_NUM_CORES = 2
_NUM_SUBCORES = 16
_NW = _NUM_CORES * _NUM_SUBCORES

_CHUNK = 16


@functools.cache
def _make_gather(B: int, D: int):
    assert B % _NW == 0
    rows_per_w = B // _NW
    assert rows_per_w % _CHUNK == 0
    num_chunks = rows_per_w // _CHUNK
    mesh = plsc.VectorSubcoreMesh(core_axis_name="c", subcore_axis_name="s")

    @functools.partial(
        pl.kernel,
        mesh=mesh,
        out_type=jax.ShapeDtypeStruct((B, D), jnp.float32),
        scratch_types=[
            pltpu.VMEM((rows_per_w,), jnp.int32),
            pltpu.VMEM((_CHUNK, D), jnp.float32),
            pltpu.SemaphoreType.DMA,
        ],
    )
    def gather_kernel(table_hbm, idx_hbm, out_hbm, idx_v, rows_v, gsem):
        wid = lax.axis_index("s") * _NUM_CORES + lax.axis_index("c")
        base = wid * rows_per_w
        pltpu.sync_copy(idx_hbm.at[pl.ds(base, rows_per_w)], idx_v)

        def body(g, carry):
            pltpu.async_copy(
                table_hbm.at[idx_v.at[pl.ds(g * _CHUNK, _CHUNK)]], rows_v, gsem
            ).wait()
            pltpu.sync_copy(rows_v, out_hbm.at[pl.ds(base + g * _CHUNK, _CHUNK)])
            return carry

        lax.fori_loop(0, num_chunks, body, 0)

    return gather_kernel


def kernel(inputs):
    B, D = inputs.shape
    idx = jax.random.permutation(jax.random.key(42), B).astype(jnp.int32)
    return _make_gather(B, D)(inputs, idx)


# speedup vs baseline: 1.6220x; 1.1581x over previous
import functools

import jax
import jax.numpy as jnp
import numpy as np
from jax import lax
from jax.experimental import pallas as pl
from jax.experimental.pallas import tpu as pltpu
from jax.experimental.pallas import tpu_sc as plsc

_NUM_CORES = 2
_NUM_SUBCORES = 16
_NW = _NUM_CORES * _NUM_SUBCORES

_CHUNK = 16


@functools.cache
def _make_gather(B: int, D: int):
    assert B % _NW == 0
    rows_per_w = B // _NW
    assert rows_per_w % _CHUNK == 0
    num_chunks = rows_per_w // _CHUNK
    mesh = plsc.VectorSubcoreMesh(core_axis_name="c", subcore_axis_name="s")

    assert num_chunks >= 4 and num_chunks % 2 == 0

    @functools.partial(
        pl.kernel,
        mesh=mesh,
        out_type=jax.ShapeDtypeStruct((B, D), jnp.float32),
        scratch_types=[
            pltpu.VMEM((rows_per_w,), jnp.int32),
            pltpu.VMEM((_CHUNK, D), jnp.float32),
            pltpu.VMEM((_CHUNK, D), jnp.float32),
            pltpu.SemaphoreType.DMA,
            pltpu.SemaphoreType.DMA,
            pltpu.SemaphoreType.DMA,
            pltpu.SemaphoreType.DMA,
        ],
    )
    def gather_kernel(
        table_hbm, idx_hbm, out_hbm, idx_v, rows0, rows1, gsem0, gsem1, ssem0, ssem1
    ):
        wid = lax.axis_index("s") * _NUM_CORES + lax.axis_index("c")
        base = wid * rows_per_w
        pltpu.sync_copy(idx_hbm.at[pl.ds(base, rows_per_w)], idx_v)

        bufs = (rows0, rows1)
        gsems = (gsem0, gsem1)
        ssems = (ssem0, ssem1)

        def gather_copy(g, buf, sem):
            return pltpu.make_async_copy(
                table_hbm.at[idx_v.at[pl.ds(g * _CHUNK, _CHUNK)]], buf, sem
            )

        def store_copy(g, buf, sem):
            return pltpu.make_async_copy(
                buf, out_hbm.at[pl.ds(base + g * _CHUNK, _CHUNK)], sem
            )

        gather_copy(0, bufs[0], gsems[0]).start()
        gather_copy(1, bufs[1], gsems[1]).start()

        def steady(i, carry):
            for b in range(2):
                g = 2 * i + b
                gather_copy(g, bufs[b], gsems[b]).wait()
                store_copy(g, bufs[b], ssems[b]).start()
                store_copy(g, bufs[b], ssems[b]).wait()
                gather_copy(g + 2, bufs[b], gsems[b]).start()
            return carry

        lax.fori_loop(0, num_chunks // 2 - 1, steady, 0)

        for b in range(2):
            g = num_chunks - 2 + b
            gather_copy(g, bufs[b], gsems[b]).wait()
            store_copy(g, bufs[b], ssems[b]).start()
        for b in range(2):
            g = num_chunks - 2 + b
            store_copy(g, bufs[b], ssems[b]).wait()

    return gather_kernel


def kernel(inputs):
    B, D = inputs.shape
    idx = jax.random.permutation(jax.random.key(42), B).astype(jnp.int32)
    return _make_gather(B, D)(inputs, idx)


# speedup vs baseline: 1.6361x; 1.0087x over previous
import functools

import jax
import jax.numpy as jnp
import numpy as np
from jax import lax
from jax.experimental import pallas as pl
from jax.experimental.pallas import tpu as pltpu
from jax.experimental.pallas import tpu_sc as plsc

_NUM_CORES = 2
_NUM_SUBCORES = 16
_NW = _NUM_CORES * _NUM_SUBCORES

_CHUNK = 8
_NBUF = 4


@functools.cache
def _make_gather(B: int, D: int):
    assert B % _NW == 0
    rows_per_w = B // _NW
    assert rows_per_w % _CHUNK == 0
    num_chunks = rows_per_w // _CHUNK
    mesh = plsc.VectorSubcoreMesh(core_axis_name="c", subcore_axis_name="s")

    assert num_chunks % _NBUF == 0 and num_chunks // _NBUF >= 2

    @functools.partial(
        pl.kernel,
        mesh=mesh,
        out_type=jax.ShapeDtypeStruct((B, D), jnp.float32),
        scratch_types=[
            pltpu.VMEM((rows_per_w,), jnp.int32),
        ]
        + [pltpu.VMEM((_CHUNK, D), jnp.float32)] * _NBUF
        + [pltpu.SemaphoreType.DMA] * (2 * _NBUF),
    )
    def gather_kernel(table_hbm, idx_hbm, out_hbm, idx_v, *bufs_and_sems):
        bufs = bufs_and_sems[:_NBUF]
        gsems = bufs_and_sems[_NBUF : 2 * _NBUF]
        ssems = bufs_and_sems[2 * _NBUF :]
        wid = lax.axis_index("s") * _NUM_CORES + lax.axis_index("c")
        base = wid * rows_per_w
        pltpu.sync_copy(idx_hbm.at[pl.ds(base, rows_per_w)], idx_v)

        def gather_copy(g, b):
            return pltpu.make_async_copy(
                table_hbm.at[idx_v.at[pl.ds(g * _CHUNK, _CHUNK)]], bufs[b], gsems[b]
            )

        def store_copy(g, b):
            return pltpu.make_async_copy(
                bufs[b], out_hbm.at[pl.ds(base + g * _CHUNK, _CHUNK)], ssems[b]
            )

        for b in range(_NBUF):
            gather_copy(b, b).start()

        def steady(i, carry):
            for b in range(_NBUF):
                g = _NBUF * i + b
                gather_copy(g, b).wait()
                store_copy(g, b).start()
                store_copy(g, b).wait()
                gather_copy(g + _NBUF, b).start()
            return carry

        lax.fori_loop(0, num_chunks // _NBUF - 1, steady, 0)

        for b in range(_NBUF):
            g = num_chunks - _NBUF + b
            gather_copy(g, b).wait()
            store_copy(g, b).start()
        for b in range(_NBUF):
            g = num_chunks - _NBUF + b
            store_copy(g, b).wait()

    return gather_kernel


def kernel(inputs):
    B, D = inputs.shape
    idx = jax.random.permutation(jax.random.key(42), B).astype(jnp.int32)
    return _make_gather(B, D)(inputs, idx)


# speedup vs baseline: 2.0154x; 1.2319x over previous
import functools

import jax
import jax.numpy as jnp
import numpy as np
from jax import lax
from jax.experimental import pallas as pl
from jax.experimental.pallas import tpu as pltpu
from jax.experimental.pallas import tpu_sc as plsc

_NUM_CORES = 2
_NUM_SUBCORES = 16
_NW = _NUM_CORES * _NUM_SUBCORES

_CHUNK = 8
_NBUF = 4


@functools.cache
def _make_scatter(B: int, D: int):
    assert B % _NW == 0
    rows_per_w = B // _NW
    assert rows_per_w % _CHUNK == 0
    num_chunks = rows_per_w // _CHUNK
    mesh = plsc.VectorSubcoreMesh(core_axis_name="c", subcore_axis_name="s")

    assert num_chunks % _NBUF == 0 and num_chunks // _NBUF >= 2

    @functools.partial(
        pl.kernel,
        mesh=mesh,
        out_type=jax.ShapeDtypeStruct((B, D), jnp.float32),
        scratch_types=[
            pltpu.VMEM((num_chunks, _CHUNK), jnp.int32),
        ]
        + [pltpu.VMEM((_CHUNK, D), jnp.float32)] * _NBUF
        + [pltpu.SemaphoreType.DMA] * (2 * _NBUF),
    )
    def scatter_kernel(table_hbm, idx_hbm, out_hbm, idx_v, *bufs_and_sems):
        bufs = bufs_and_sems[:_NBUF]
        gsems = bufs_and_sems[_NBUF : 2 * _NBUF]
        ssems = bufs_and_sems[2 * _NBUF :]
        wid = lax.axis_index("s") * _NUM_CORES + lax.axis_index("c")
        base = wid * rows_per_w
        pltpu.sync_copy(idx_hbm.at[pl.ds(wid * num_chunks, num_chunks)], idx_v)

        def gather_copy(g, b):
            return pltpu.make_async_copy(
                table_hbm.at[pl.ds(base + g * _CHUNK, _CHUNK)], bufs[b], gsems[b]
            )

        def store_copy(g, b):
            return pltpu.make_async_copy(
                bufs[b], out_hbm.at[idx_v.at[g]], ssems[b]
            )

        gather_copy(0, 0).start()
        gather_copy(1, 1).start()
        for g in (0, 1):
            gather_copy(g, g % _NBUF).wait()
            store_copy(g, g % _NBUF).start()
            gather_copy(g + 2, (g + 2) % _NBUF).start()

        def steady(i, carry):
            for j in range(_NBUF):
                g = _NBUF * i + 2 + j
                b = (2 + j) % _NBUF
                gather_copy(g, b).wait()
                store_copy(g, b).start()
                store_copy(g - 2, (b + 2) % _NBUF).wait()
                gather_copy(g + 2, (b + 2) % _NBUF).start()
            return carry

        lax.fori_loop(0, (num_chunks - 4) // _NBUF, steady, 0)

        for g in (num_chunks - 2, num_chunks - 1):
            b = g % _NBUF
            gather_copy(g, b).wait()
            store_copy(g, b).start()
            store_copy(g - 2, (b + 2) % _NBUF).wait()
        for g in (num_chunks - 2, num_chunks - 1):
            store_copy(g, g % _NBUF).wait()

    return scatter_kernel


@functools.cache
def _perm(n: int) -> np.ndarray:
    with jax.ensure_compile_time_eval():
        return np.asarray(
            jax.random.permutation(jax.random.key(42), n), dtype=np.int32
        )


def kernel(inputs):
    B, D = inputs.shape
    perm = _perm(B)
    inv = np.empty_like(perm)
    inv[perm] = np.arange(B, dtype=np.int32)
    idx = jnp.asarray(inv.reshape(B // _CHUNK, _CHUNK))
    return _make_scatter(B, D)(inputs, idx)


# speedup vs baseline: 2.0610x; 1.0226x over previous
import functools

import jax
import jax.numpy as jnp
import numpy as np
from jax import lax
from jax.experimental import pallas as pl
from jax.experimental.pallas import tpu as pltpu
from jax.experimental.pallas import tpu_sc as plsc

_NUM_CORES = 2
_NUM_SUBCORES = 16
_NW = _NUM_CORES * _NUM_SUBCORES

_CHUNK = 8
_NBUF = 4


@functools.cache
def _make_gather(B: int, D: int):
    assert B % _NW == 0
    rows_per_w = B // _NW
    assert rows_per_w % _CHUNK == 0
    num_chunks = rows_per_w // _CHUNK
    mesh = plsc.VectorSubcoreMesh(core_axis_name="c", subcore_axis_name="s")

    assert num_chunks % _NBUF == 0 and num_chunks // _NBUF >= 2

    @functools.partial(
        pl.kernel,
        mesh=mesh,
        out_type=jax.ShapeDtypeStruct((B, D), jnp.float32),
        scratch_types=[
            pltpu.VMEM((rows_per_w,), jnp.int32),
        ]
        + [pltpu.VMEM((_CHUNK, D), jnp.float32)] * _NBUF
        + [pltpu.SemaphoreType.DMA] * (2 * _NBUF),
    )
    def gather_kernel(table_hbm, idx_hbm, out_hbm, idx_v, *bufs_and_sems):
        bufs = bufs_and_sems[:_NBUF]
        gsems = bufs_and_sems[_NBUF : 2 * _NBUF]
        ssems = bufs_and_sems[2 * _NBUF :]
        wid = lax.axis_index("s") * _NUM_CORES + lax.axis_index("c")
        base = wid * rows_per_w
        pltpu.sync_copy(idx_hbm.at[pl.ds(base, rows_per_w)], idx_v)

        def gather_copy(g, b):
            return pltpu.make_async_copy(
                table_hbm.at[idx_v.at[pl.ds(g * _CHUNK, _CHUNK)]], bufs[b], gsems[b]
            )

        def store_copy(g, b):
            return pltpu.make_async_copy(
                bufs[b], out_hbm.at[pl.ds(base + g * _CHUNK, _CHUNK)], ssems[b]
            )

        gather_copy(0, 0).start()
        gather_copy(1, 1).start()
        for g in (0, 1):
            gather_copy(g, g % _NBUF).wait()
            store_copy(g, g % _NBUF).start()
            gather_copy(g + 2, (g + 2) % _NBUF).start()

        def steady(i, carry):
            for j in range(_NBUF):
                g = _NBUF * i + 2 + j
                b = (2 + j) % _NBUF
                gather_copy(g, b).wait()
                store_copy(g, b).start()
                store_copy(g - 2, (b + 2) % _NBUF).wait()
                gather_copy(g + 2, (b + 2) % _NBUF).start()
            return carry

        lax.fori_loop(0, (num_chunks - 4) // _NBUF, steady, 0)

        for g in (num_chunks - 2, num_chunks - 1):
            b = g % _NBUF
            gather_copy(g, b).wait()
            store_copy(g, b).start()
            store_copy(g - 2, (b + 2) % _NBUF).wait()
        for g in (num_chunks - 2, num_chunks - 1):
            store_copy(g, g % _NBUF).wait()

    return gather_kernel


@functools.cache
def _perm(n: int) -> np.ndarray:
    with jax.ensure_compile_time_eval():
        return np.asarray(
            jax.random.permutation(jax.random.key(42), n), dtype=np.int32
        )


def kernel(inputs):
    B, D = inputs.shape
    idx = jnp.asarray(_perm(B))
    return _make_gather(B, D)(inputs, idx)
